# baseline (speedup 1.0000x reference)
import functools

import jax
import jax.numpy as jnp
from jax.experimental import pallas as pl
from jax.experimental.pallas import tpu as pltpu

_B, _N, _K = 8, 2048, 16
_EMBED, _HID, _NCLS = 64, 128, 50
_EPS = 1e-5
_BIG = 1e30
_F32 = jnp.float32


def _mm(a, b):
    return jax.lax.dot_general(a, b, (((1,), (0,)), ((), ())),
                               preferred_element_type=_F32)


def _bn(x, g, b):
    mu = jnp.mean(x, axis=0, keepdims=True)
    var = jnp.mean((x - mu) ** 2, axis=0, keepdims=True)
    return (x - mu) / jnp.sqrt(var + _EPS) * g + b


def _embed_kernel(pos_ref, w0_ref, b0_ref, g0_ref, bb0_ref,
                  w1_ref, b1_ref, g1_ref, bb1_ref, out_ref):
    h = _mm(pos_ref[...], w0_ref[...]) + b0_ref[...]
    h = jnp.maximum(_bn(h, g0_ref[...], bb0_ref[...]), 0.0)
    h = _mm(h, w1_ref[...]) + b1_ref[...]
    out_ref[...] = jnp.maximum(_bn(h, g1_ref[...], bb1_ref[...]), 0.0)


_TR = 256


def _knn_kernel(rows_ref, cols_ref, m_ref):
    t = pl.program_id(1)
    pr = rows_ref[...]
    pc = cols_ref[...]
    cross = jax.lax.dot_general(pr, pc, (((1,), (1,)), ((), ())),
                                preferred_element_type=_F32)
    sq_col = jnp.sum(pr * pr, axis=1, keepdims=True)
    sq_row = jax.lax.dot_general(jnp.ones((1, 8), _F32), pc * pc,
                                 (((1,), (1,)), ((), ())),
                                 preferred_element_type=_F32,
                                 precision=jax.lax.Precision.HIGHEST)
    d = sq_col + sq_row - 2.0 * cross
    r = jax.lax.broadcasted_iota(jnp.int32, (_TR, _N), 0) + t * _TR
    c = jax.lax.broadcasted_iota(jnp.int32, (_TR, _N), 1)
    d = jnp.where(r == c, d + 1e10, d)
    m = jnp.min(d, axis=1, keepdims=True)
    for _ in range(_K):
        first = jnp.min(jnp.where(d == m, c, _N), axis=1, keepdims=True)
        d = jnp.where(c == first, _BIG, d)
        m = jnp.min(d, axis=1, keepdims=True)
    m_ref[...] = (d >= 1e20).astype(jnp.int8)


def _conv_kernel(h_ref, pos_ref, m_ref, sc_ref, sh_ref,
                 wl_ref, ws_ref, wp_ref, bp_ref,
                 out_ref, ps_ref, pq_ref):
    b = pl.program_id(0)
    x = jnp.maximum(h_ref[...] * sc_ref[...] + sh_ref[...], 0.0)
    v = _mm(x, wl_ref[...])
    a = _mm(x, ws_ref[...])
    q = _mm(pos_ref[...], wp_ref[...])
    t = -(a + q)
    e = jnp.exp(t - jnp.max(t, axis=0, keepdims=True))
    ev = (v - q + bp_ref[...]) * e
    mm_f = m_ref[...].astype(_F32)
    den = _mm(mm_f, e)
    num = _mm(mm_f, ev)
    out = num / den + q
    out_ref[...] = out

    @pl.when(b == 0)
    def _init():
        ps_ref[...] = jnp.zeros_like(ps_ref)
        pq_ref[...] = jnp.zeros_like(pq_ref)

    ps_ref[...] += jnp.sum(out, axis=0, keepdims=True)
    pq_ref[...] += jnp.sum(out * out, axis=0, keepdims=True)


def _dec_kernel(h_ref, sc_ref, sh_ref, w0_ref, b0_ref, g_ref, bb_ref,
                w1_ref, b1_ref, out_ref):
    x = jnp.maximum(h_ref[...] * sc_ref[...] + sh_ref[...], 0.0)
    y = _mm(x, w0_ref[...]) + b0_ref[...]
    z = jnp.maximum(_bn(y, g_ref[...], bb_ref[...]), 0.0)
    out_ref[...] = _mm(z, w1_ref[...]) + b1_ref[...]


def _affine_from_stats(ps, pq, g, b):
    n = float(_B * _N)
    mu = ps / n
    var = pq / n - mu * mu
    s = g[None, :] / jnp.sqrt(var + _EPS)
    return s, b[None, :] - mu * s


def _conv_layer(h3, pos3, m8, scale, shift, p, i, cin):
    wl = p[f'pt{i}_lin'].T
    ws = p[f'pt{i}_src'].T
    wp = jnp.pad(p[f'pt{i}_posW'].T, ((0, 5), (0, 0)))
    bp = p[f'pt{i}_posb'][None, :]
    bcast = lambda b_: (0, 0)
    out, ps, pq = pl.pallas_call(
        _conv_kernel,
        grid=(_B,),
        in_specs=[
            pl.BlockSpec((None, _N, cin), lambda b_: (b_, 0, 0)),
            pl.BlockSpec((None, _N, 8), lambda b_: (b_, 0, 0)),
            pl.BlockSpec((None, _N, _N), lambda b_: (b_, 0, 0)),
            pl.BlockSpec((1, cin), bcast),
            pl.BlockSpec((1, cin), bcast),
            pl.BlockSpec((cin, _HID), bcast),
            pl.BlockSpec((cin, _HID), bcast),
            pl.BlockSpec((8, _HID), bcast),
            pl.BlockSpec((1, _HID), bcast),
        ],
        out_specs=[
            pl.BlockSpec((None, _N, _HID), lambda b_: (b_, 0, 0)),
            pl.BlockSpec((1, _HID), bcast),
            pl.BlockSpec((1, _HID), bcast),
        ],
        out_shape=[
            jax.ShapeDtypeStruct((_B, _N, _HID), _F32),
            jax.ShapeDtypeStruct((1, _HID), _F32),
            jax.ShapeDtypeStruct((1, _HID), _F32),
        ],
    )(h3, pos3, m8, scale, shift, wl, ws, wp, bp)
    return out, ps[0], pq[0]


def kernel(points, params):
    p = params
    pos = jnp.pad(points.reshape(_B * _N, 3), ((0, 0), (0, 5)))
    pos3 = pos.reshape(_B, _N, 8)

    h_embed = pl.pallas_call(
        _embed_kernel,
        out_shape=jax.ShapeDtypeStruct((_B * _N, _EMBED), _F32),
    )(pos,
      jnp.pad(p['embed_W0'].T, ((0, 5), (0, 0))), p['embed_b0'][None, :],
      p['embed_bn_g'][None, :], p['embed_bn_b'][None, :],
      p['embed_W1'].T, p['embed_b1'][None, :],
      p['bn_embed_g'][None, :], p['bn_embed_b'][None, :])

    m8 = pl.pallas_call(
        _knn_kernel,
        grid=(_B, _N // _TR),
        in_specs=[pl.BlockSpec((None, _TR, 8), lambda b_, t_: (b_, t_, 0)),
                  pl.BlockSpec((None, _N, 8), lambda b_, t_: (b_, 0, 0))],
        out_specs=pl.BlockSpec((None, _TR, _N), lambda b_, t_: (b_, t_, 0)),
        out_shape=jax.ShapeDtypeStruct((_B, _N, _N), jnp.int8),
        compiler_params=pltpu.CompilerParams(
            dimension_semantics=("parallel", "parallel")),
    )(pos3, pos3)

    ones = jnp.ones((1, _EMBED), _F32)
    zeros = jnp.zeros((1, _EMBED), _F32)
    h3 = h_embed.reshape(_B, _N, _EMBED)
    c0, ps0, pq0 = _conv_layer(h3, pos3, m8, ones, zeros, p, 0, _EMBED)
    s1, t1 = _affine_from_stats(ps0, pq0, p['pt0_bn_g'], p['pt0_bn_b'])
    c1, ps1, pq1 = _conv_layer(c0, pos3, m8, s1, t1, p, 1, _HID)
    s2, t2 = _affine_from_stats(ps1, pq1, p['pt1_bn_g'], p['pt1_bn_b'])

    w1d = jnp.pad(p['dec_W1'].T, ((0, 0), (0, 64 - _NCLS)))
    b1d = jnp.pad(p['dec_b1'], (0, 64 - _NCLS))[None, :]
    logits = pl.pallas_call(
        _dec_kernel,
        out_shape=jax.ShapeDtypeStruct((_B * _N, 64), _F32),
    )(c1.reshape(_B * _N, _HID), s2, t2,
      p['dec_W0'].T, p['dec_b0'][None, :],
      p['dec_bn_g'][None, :], p['dec_bn_b'][None, :],
      w1d, b1d)
    return logits[:, :_NCLS].reshape(_B, _N, _NCLS)


# speedup vs baseline: 1.1767x; 1.1767x over previous
import functools

import jax
import jax.numpy as jnp
from jax.experimental import pallas as pl
from jax.experimental.pallas import tpu as pltpu

_B, _N, _K = 8, 2048, 16
_EMBED, _HID, _NCLS = 64, 128, 50
_EPS = 1e-5
_BIG = 1e30
_F32 = jnp.float32


def _mm(a, b):
    return jax.lax.dot_general(a, b, (((1,), (0,)), ((), ())),
                               preferred_element_type=_F32)


def _bn(x, g, b):
    mu = jnp.mean(x, axis=0, keepdims=True)
    var = jnp.mean((x - mu) ** 2, axis=0, keepdims=True)
    return (x - mu) / jnp.sqrt(var + _EPS) * g + b


def _embed_kernel(pos_ref, w0_ref, b0_ref, g0_ref, bb0_ref,
                  w1_ref, b1_ref, g1_ref, bb1_ref, out_ref):
    h = _mm(pos_ref[...], w0_ref[...]) + b0_ref[...]
    h = jnp.maximum(_bn(h, g0_ref[...], bb0_ref[...]), 0.0)
    h = _mm(h, w1_ref[...]) + b1_ref[...]
    out_ref[...] = jnp.maximum(_bn(h, g1_ref[...], bb1_ref[...]), 0.0)


_TR = 256


def _knn_kernel(rows_ref, cols_ref, m_ref):
    t = pl.program_id(1)
    pr = rows_ref[...]
    pc = cols_ref[...]
    cross = jax.lax.dot_general(pr, pc, (((1,), (1,)), ((), ())),
                                preferred_element_type=_F32)
    sq_col = jnp.sum(pr * pr, axis=1, keepdims=True)
    sq_row = jax.lax.dot_general(jnp.ones((1, 8), _F32), pc * pc,
                                 (((1,), (1,)), ((), ())),
                                 preferred_element_type=_F32,
                                 precision=jax.lax.Precision.HIGHEST)
    d = sq_col + sq_row - 2.0 * cross
    r = jax.lax.broadcasted_iota(jnp.int32, (_TR, _N), 0) + t * _TR
    c = jax.lax.broadcasted_iota(jnp.int32, (_TR, _N), 1)
    d = jnp.where(r == c, d + 1e10, d)
    for _ in range(_K):
        first = jnp.argmin(d, axis=1).astype(jnp.int32)[:, None]
        d = jnp.where(c == first, _BIG, d)
    m_ref[...] = (d >= 1e20).astype(jnp.int8)


def _conv_kernel(h_ref, pos_ref, m_ref, sc_ref, sh_ref,
                 wl_ref, ws_ref, wp_ref, bp_ref,
                 out_ref, ps_ref, pq_ref):
    b = pl.program_id(0)
    x = jnp.maximum(h_ref[...] * sc_ref[...] + sh_ref[...], 0.0)
    v = _mm(x, wl_ref[...])
    a = _mm(x, ws_ref[...])
    q = _mm(pos_ref[...], wp_ref[...])
    t = -(a + q)
    e = jnp.exp(t - jnp.max(t, axis=0, keepdims=True))
    ev = (v - q + bp_ref[...]) * e
    mm_f = m_ref[...].astype(_F32)
    den = _mm(mm_f, e)
    num = _mm(mm_f, ev)
    out = num / den + q
    out_ref[...] = out

    @pl.when(b == 0)
    def _init():
        ps_ref[...] = jnp.zeros_like(ps_ref)
        pq_ref[...] = jnp.zeros_like(pq_ref)

    ps_ref[...] += jnp.sum(out, axis=0, keepdims=True)
    pq_ref[...] += jnp.sum(out * out, axis=0, keepdims=True)


def _dec_kernel(h_ref, sc_ref, sh_ref, w0_ref, b0_ref, g_ref, bb_ref,
                w1_ref, b1_ref, out_ref):
    x = jnp.maximum(h_ref[...] * sc_ref[...] + sh_ref[...], 0.0)
    y = _mm(x, w0_ref[...]) + b0_ref[...]
    z = jnp.maximum(_bn(y, g_ref[...], bb_ref[...]), 0.0)
    out_ref[...] = _mm(z, w1_ref[...]) + b1_ref[...]


def _affine_from_stats(ps, pq, g, b):
    n = float(_B * _N)
    mu = ps / n
    var = pq / n - mu * mu
    s = g[None, :] / jnp.sqrt(var + _EPS)
    return s, b[None, :] - mu * s


def _conv_layer(h3, pos3, m8, scale, shift, p, i, cin):
    wl = p[f'pt{i}_lin'].T
    ws = p[f'pt{i}_src'].T
    wp = jnp.pad(p[f'pt{i}_posW'].T, ((0, 5), (0, 0)))
    bp = p[f'pt{i}_posb'][None, :]
    bcast = lambda b_: (0, 0)
    out, ps, pq = pl.pallas_call(
        _conv_kernel,
        grid=(_B,),
        in_specs=[
            pl.BlockSpec((None, _N, cin), lambda b_: (b_, 0, 0)),
            pl.BlockSpec((None, _N, 8), lambda b_: (b_, 0, 0)),
            pl.BlockSpec((None, _N, _N), lambda b_: (b_, 0, 0)),
            pl.BlockSpec((1, cin), bcast),
            pl.BlockSpec((1, cin), bcast),
            pl.BlockSpec((cin, _HID), bcast),
            pl.BlockSpec((cin, _HID), bcast),
            pl.BlockSpec((8, _HID), bcast),
            pl.BlockSpec((1, _HID), bcast),
        ],
        out_specs=[
            pl.BlockSpec((None, _N, _HID), lambda b_: (b_, 0, 0)),
            pl.BlockSpec((1, _HID), bcast),
            pl.BlockSpec((1, _HID), bcast),
        ],
        out_shape=[
            jax.ShapeDtypeStruct((_B, _N, _HID), _F32),
            jax.ShapeDtypeStruct((1, _HID), _F32),
            jax.ShapeDtypeStruct((1, _HID), _F32),
        ],
    )(h3, pos3, m8, scale, shift, wl, ws, wp, bp)
    return out, ps[0], pq[0]


def kernel(points, params):
    p = params
    pos = jnp.pad(points.reshape(_B * _N, 3), ((0, 0), (0, 5)))
    pos3 = pos.reshape(_B, _N, 8)

    h_embed = pl.pallas_call(
        _embed_kernel,
        out_shape=jax.ShapeDtypeStruct((_B * _N, _EMBED), _F32),
    )(pos,
      jnp.pad(p['embed_W0'].T, ((0, 5), (0, 0))), p['embed_b0'][None, :],
      p['embed_bn_g'][None, :], p['embed_bn_b'][None, :],
      p['embed_W1'].T, p['embed_b1'][None, :],
      p['bn_embed_g'][None, :], p['bn_embed_b'][None, :])

    m8 = pl.pallas_call(
        _knn_kernel,
        grid=(_B, _N // _TR),
        in_specs=[pl.BlockSpec((None, _TR, 8), lambda b_, t_: (b_, t_, 0)),
                  pl.BlockSpec((None, _N, 8), lambda b_, t_: (b_, 0, 0))],
        out_specs=pl.BlockSpec((None, _TR, _N), lambda b_, t_: (b_, t_, 0)),
        out_shape=jax.ShapeDtypeStruct((_B, _N, _N), jnp.int8),
        compiler_params=pltpu.CompilerParams(
            dimension_semantics=("parallel", "parallel")),
    )(pos3, pos3)

    ones = jnp.ones((1, _EMBED), _F32)
    zeros = jnp.zeros((1, _EMBED), _F32)
    h3 = h_embed.reshape(_B, _N, _EMBED)
    c0, ps0, pq0 = _conv_layer(h3, pos3, m8, ones, zeros, p, 0, _EMBED)
    s1, t1 = _affine_from_stats(ps0, pq0, p['pt0_bn_g'], p['pt0_bn_b'])
    c1, ps1, pq1 = _conv_layer(c0, pos3, m8, s1, t1, p, 1, _HID)
    s2, t2 = _affine_from_stats(ps1, pq1, p['pt1_bn_g'], p['pt1_bn_b'])

    w1d = jnp.pad(p['dec_W1'].T, ((0, 0), (0, 64 - _NCLS)))
    b1d = jnp.pad(p['dec_b1'], (0, 64 - _NCLS))[None, :]
    logits = pl.pallas_call(
        _dec_kernel,
        out_shape=jax.ShapeDtypeStruct((_B * _N, 64), _F32),
    )(c1.reshape(_B * _N, _HID), s2, t2,
      p['dec_W0'].T, p['dec_b0'][None, :],
      p['dec_bn_g'][None, :], p['dec_bn_b'][None, :],
      w1d, b1d)
    return logits[:, :_NCLS].reshape(_B, _N, _NCLS)
